# Initial kernel scaffold; baseline (speedup 1.0000x reference)
import jax
import jax.numpy as jnp
from jax import lax
from jax.experimental import pallas as pl
from jax.experimental.pallas import tpu as pltpu
from jax.experimental.pallas import tpu_sc as plsc

N = 10000
E = 320000
D = 128
G = 64
RD = 256
NG = N * G
NC = 2
NS = 16
NW = NC * NS
EPT = E // NW
ROWS = (EPT + 127) // 128
EPAD = ROWS * 128
WPT = NG // NS
ZCH = 8000


def _sc_body(src_hbm, dst_hbm, attr_hbm, batch_hbm, out_hbm,
             batch_v, src_v, dst_v, attr_v, keys_v, vals_v, zbuf_v, w_sh):
    c = lax.axis_index("c")
    s = lax.axis_index("s")
    wid = c * NS + s
    ebase = wid * EPT

    pltpu.sync_copy(batch_hbm, batch_v)
    pltpu.sync_copy(src_hbm.at[pl.ds(ebase, EPT)], src_v.at[pl.ds(0, EPT)])
    pltpu.sync_copy(dst_hbm.at[pl.ds(ebase, EPT)], dst_v.at[pl.ds(0, EPT)])
    pltpu.sync_copy(attr_hbm.at[pl.ds(ebase, EPT)], attr_v.at[pl.ds(0, EPT)])

    def zero_body(i, carry):
        zbuf_v[pl.ds(i * 16, 16)] = jnp.zeros((16,), jnp.float32)
        return carry
    lax.fori_loop(0, ZCH // 16, zero_body, 0)
    for r in range(WPT // ZCH):
        pltpu.sync_copy(zbuf_v, w_sh.at[pl.ds(s * WPT + r * ZCH, ZCH)])

    lanes = lax.iota(jnp.int32, 16)

    def key_body(j, carry):
        for k in range(8):
            off = j * 128 + k * 16
            valid = (off + lanes) < EPT
            d = jnp.where(valid, dst_v[pl.ds(off, 16)], 0)
            sg = plsc.load_gather(batch_v, [d])
            sv = src_v[pl.ds(off, 16)]
            key = jnp.where(valid, sv * G + sg, 0)
            a = jnp.where(valid, attr_v[pl.ds(off, 16)], 0.0)
            keys_v[j, pl.ds(k * 16, 16)] = key
            vals_v[j, pl.ds(k * 16, 16)] = a
        return carry
    lax.fori_loop(0, ROWS, key_body, 0)

    plsc.subcore_barrier()

    def scat_body(j, carry):
        pltpu.sync_copy(vals_v.at[j], w_sh.at[keys_v.at[j]], add=True)
        return carry
    lax.fori_loop(0, ROWS, scat_body, 0)

    plsc.subcore_barrier()

    for r in range(WPT // ZCH):
        off = s * WPT + r * ZCH
        pltpu.sync_copy(w_sh.at[pl.ds(off, ZCH)], zbuf_v)
        pltpu.sync_copy(zbuf_v, out_hbm.at[pl.ds(c * NG + off, ZCH)])


def _build_w(src, dst, attr, batch):
    mesh = plsc.VectorSubcoreMesh(core_axis_name="c", subcore_axis_name="s")
    f = pl.kernel(
        _sc_body,
        out_type=jax.ShapeDtypeStruct((NC * NG,), jnp.float32),
        mesh=mesh,
        scratch_types=[
            pltpu.VMEM((N,), jnp.int32),
            pltpu.VMEM((EPAD,), jnp.int32),
            pltpu.VMEM((EPAD,), jnp.int32),
            pltpu.VMEM((EPAD,), jnp.float32),
            pltpu.VMEM((ROWS, 128), jnp.int32),
            pltpu.VMEM((ROWS, 128), jnp.float32),
            pltpu.VMEM((ZCH,), jnp.float32),
            pltpu.VMEM_SHARED((NG,), jnp.float32),
        ],
    )
    return f(src, dst, attr, batch)


def _bn(z, g, b):
    mu = jnp.mean(z, axis=0, keepdims=True)
    var = jnp.mean((z - mu) * (z - mu), axis=0, keepdims=True)
    return (z - mu) * jax.lax.rsqrt(var + 1e-5) * g + b


def _tc_body(w2_ref, x_ref, batchT_ref,
             fc1_W_ref, fc1_b_ref, bn1_g_ref, bn1_b_ref,
             fc2_W_ref, fc2_b_ref, bn2_g_ref, bn2_b_ref,
             r1_W_ref, r1_b_ref, rbn1_g_ref, rbn1_b_ref,
             r2_W_ref, r2_b_ref, rbn2_g_ref, rbn2_b_ref,
             out_WT_ref, out_b_ref, o_ref):
    hi = jax.lax.Precision.HIGHEST
    w = w2_ref[0] + w2_ref[1]
    sums = jax.lax.dot_general(w, x_ref[...],
                               (((0,), (0,)), ((), ())),
                               precision=hi)
    gids = jax.lax.broadcasted_iota(jnp.int32, (G, N), 0)
    onehot = (gids == batchT_ref[...]).astype(jnp.float32)
    cnt = jnp.sum(onehot, axis=1, keepdims=True)
    p = sums / jnp.maximum(cnt, 1.0)

    h = jnp.maximum(_bn(jnp.dot(p, fc1_W_ref[...], precision=hi)
                        + fc1_b_ref[...], bn1_g_ref[...], bn1_b_ref[...]), 0.0)
    h = jnp.maximum(_bn(jnp.dot(h, fc2_W_ref[...], precision=hi)
                        + fc2_b_ref[...], bn2_g_ref[...], bn2_b_ref[...]), 0.0)
    res = h
    h = _bn(jnp.dot(h, r1_W_ref[...], precision=hi)
            + r1_b_ref[...], rbn1_g_ref[...], rbn1_b_ref[...])
    h = jnp.maximum(h, 0.0)
    h = _bn(jnp.dot(h, r2_W_ref[...], precision=hi)
            + r2_b_ref[...], rbn2_g_ref[...], rbn2_b_ref[...])
    h = jnp.maximum(h + res, 0.0)
    o_ref[...] = (jnp.sum(h * out_WT_ref[...], axis=1, keepdims=True)
                  + out_b_ref[...])


def kernel(x, edge_index, edge_attr, batch,
           fc1_W, fc1_b, bn1_g, bn1_b,
           fc2_W, fc2_b, bn2_g, bn2_b,
           r1_W, r1_b, rbn1_g, rbn1_b,
           r2_W, r2_b, rbn2_g, rbn2_b,
           out_W, out_b):
    src = edge_index[0]
    dst = edge_index[1]
    attr = edge_attr.reshape(E)
    w_flat = _build_w(src, dst, attr, batch)
    w2 = w_flat.reshape(NC, N, G)

    row = lambda v: v.reshape(1, -1)
    tc = pl.pallas_call(
        _tc_body,
        out_shape=jax.ShapeDtypeStruct((G, 1), jnp.float32),
    )
    return tc(w2, x, batch.reshape(1, N),
              fc1_W, row(fc1_b), row(bn1_g), row(bn1_b),
              fc2_W, row(fc2_b), row(bn2_g), row(bn2_b),
              r1_W, row(r1_b), row(rbn1_g), row(rbn1_b),
              r2_W, row(r2_b), row(rbn2_g), row(rbn2_b),
              out_W.reshape(1, RD), out_b.reshape(1, 1))


# speedup vs baseline: 21.9610x; 21.9610x over previous
import jax
import jax.numpy as jnp
from jax import lax
from jax.experimental import pallas as pl
from jax.experimental.pallas import tpu as pltpu
from jax.experimental.pallas import tpu_sc as plsc

N = 10000
E = 320000
D = 128
G = 64
RD = 256
NG = N * G
NC = 2
NS = 16
NW = NC * NS
EPT = E // NW
ROWS = (EPT + 127) // 128
EPAD = ROWS * 128
WPT = NG // NS
ZCH = 8000


def _sc_body(src_hbm, dst_hbm, attr_hbm, batch_hbm, out_hbm,
             batch_v, src_v, dst_v, attr_v, keys_v, vals_v, zbuf_v, w_sh):
    c = lax.axis_index("c")
    s = lax.axis_index("s")
    wid = c * NS + s
    ebase = wid * EPT

    pltpu.sync_copy(batch_hbm, batch_v)
    pltpu.sync_copy(src_hbm.at[pl.ds(ebase, EPT)], src_v.at[pl.ds(0, EPT)])
    pltpu.sync_copy(dst_hbm.at[pl.ds(ebase, EPT)], dst_v.at[pl.ds(0, EPT)])
    pltpu.sync_copy(attr_hbm.at[pl.ds(ebase, EPT)], attr_v.at[pl.ds(0, EPT)])

    def zero_body(i, carry):
        zbuf_v[pl.ds(i * 16, 16)] = jnp.zeros((16,), jnp.float32)
        return carry
    lax.fori_loop(0, ZCH // 16, zero_body, 0)
    for r in range(WPT // ZCH):
        pltpu.sync_copy(zbuf_v, w_sh.at[pl.ds(s * WPT + r * ZCH, ZCH)])

    lanes = lax.iota(jnp.int32, 16)

    def key_body(j, carry):
        for k in range(8):
            off = j * 128 + k * 16
            valid = (off + lanes) < EPT
            d = jnp.where(valid, dst_v[pl.ds(off, 16)], 0)
            sg = plsc.load_gather(batch_v, [d])
            sv = src_v[pl.ds(off, 16)]
            key = jnp.where(valid, sv * G + sg, 0)
            a = jnp.where(valid, attr_v[pl.ds(off, 16)], 0.0)
            keys_v[j, pl.ds(k * 16, 16)] = key
            vals_v[j, pl.ds(k * 16, 16)] = a
        return carry
    lax.fori_loop(0, ROWS, key_body, 0)

    plsc.subcore_barrier()

    def scat_body(j, carry):
        pltpu.sync_copy(vals_v.at[j], w_sh.at[keys_v.at[j]], add=True)
        return carry
    lax.fori_loop(0, ROWS, scat_body, 0)

    plsc.subcore_barrier()

    for r in range(WPT // ZCH):
        off = s * WPT + r * ZCH
        pltpu.sync_copy(w_sh.at[pl.ds(off, ZCH)], zbuf_v)
        pltpu.sync_copy(zbuf_v, out_hbm.at[pl.ds(c * NG + off, ZCH)])


def _build_w(src, dst, attr, batch):
    mesh = plsc.VectorSubcoreMesh(core_axis_name="c", subcore_axis_name="s")
    f = pl.kernel(
        _sc_body,
        out_type=jax.ShapeDtypeStruct((NC * NG,), jnp.float32),
        mesh=mesh,
        compiler_params=pltpu.CompilerParams(needs_layout_passes=False),
        scratch_types=[
            pltpu.VMEM((N,), jnp.int32),
            pltpu.VMEM((EPAD,), jnp.int32),
            pltpu.VMEM((EPAD,), jnp.int32),
            pltpu.VMEM((EPAD,), jnp.float32),
            pltpu.VMEM((ROWS, 128), jnp.int32),
            pltpu.VMEM((ROWS, 128), jnp.float32),
            pltpu.VMEM((ZCH,), jnp.float32),
            pltpu.VMEM_SHARED((NG,), jnp.float32),
        ],
    )
    return f(src, dst, attr, batch)


def _bn(z, g, b):
    mu = jnp.mean(z, axis=0, keepdims=True)
    var = jnp.mean((z - mu) * (z - mu), axis=0, keepdims=True)
    return (z - mu) * jax.lax.rsqrt(var + 1e-5) * g + b


def _tc_body(w2_ref, x_ref, batchT_ref,
             fc1_W_ref, fc1_b_ref, bn1_g_ref, bn1_b_ref,
             fc2_W_ref, fc2_b_ref, bn2_g_ref, bn2_b_ref,
             r1_W_ref, r1_b_ref, rbn1_g_ref, rbn1_b_ref,
             r2_W_ref, r2_b_ref, rbn2_g_ref, rbn2_b_ref,
             out_WT_ref, out_b_ref, o_ref):
    hi = jax.lax.Precision.HIGHEST
    w = w2_ref[0] + w2_ref[1]
    sums = jax.lax.dot_general(w, x_ref[...],
                               (((0,), (0,)), ((), ())),
                               precision=hi)
    gids = jax.lax.broadcasted_iota(jnp.int32, (G, N), 0)
    onehot = (gids == batchT_ref[...]).astype(jnp.float32)
    cnt = jnp.sum(onehot, axis=1, keepdims=True)
    p = sums / jnp.maximum(cnt, 1.0)

    h = jnp.maximum(_bn(jnp.dot(p, fc1_W_ref[...], precision=hi)
                        + fc1_b_ref[...], bn1_g_ref[...], bn1_b_ref[...]), 0.0)
    h = jnp.maximum(_bn(jnp.dot(h, fc2_W_ref[...], precision=hi)
                        + fc2_b_ref[...], bn2_g_ref[...], bn2_b_ref[...]), 0.0)
    res = h
    h = _bn(jnp.dot(h, r1_W_ref[...], precision=hi)
            + r1_b_ref[...], rbn1_g_ref[...], rbn1_b_ref[...])
    h = jnp.maximum(h, 0.0)
    h = _bn(jnp.dot(h, r2_W_ref[...], precision=hi)
            + r2_b_ref[...], rbn2_g_ref[...], rbn2_b_ref[...])
    h = jnp.maximum(h + res, 0.0)
    o_ref[...] = (jnp.sum(h * out_WT_ref[...], axis=1, keepdims=True)
                  + out_b_ref[...])


def kernel(x, edge_index, edge_attr, batch,
           fc1_W, fc1_b, bn1_g, bn1_b,
           fc2_W, fc2_b, bn2_g, bn2_b,
           r1_W, r1_b, rbn1_g, rbn1_b,
           r2_W, r2_b, rbn2_g, rbn2_b,
           out_W, out_b):
    src = edge_index[0]
    dst = edge_index[1]
    attr = edge_attr.reshape(E)
    w_flat = _build_w(src, dst, attr, batch)
    w2 = w_flat.reshape(NC, N, G)

    row = lambda v: v.reshape(1, -1)
    tc = pl.pallas_call(
        _tc_body,
        out_shape=jax.ShapeDtypeStruct((G, 1), jnp.float32),
    )
    return tc(w2, x, batch.reshape(1, N),
              fc1_W, row(fc1_b), row(bn1_g), row(bn1_b),
              fc2_W, row(fc2_b), row(bn2_g), row(bn2_b),
              r1_W, row(r1_b), row(rbn1_g), row(rbn1_b),
              r2_W, row(r2_b), row(rbn2_g), row(rbn2_b),
              out_W.reshape(1, RD), out_b.reshape(1, 1))


# speedup vs baseline: 28.2776x; 1.2876x over previous
import jax
import jax.numpy as jnp
from jax import lax
from jax.experimental import pallas as pl
from jax.experimental.pallas import tpu as pltpu
from jax.experimental.pallas import tpu_sc as plsc

N = 10000
E = 320000
D = 128
G = 64
RD = 256
NG = N * G
NC = 2
NS = 16
NW = NC * NS
EPT = E // NW
ROWS = (EPT + 127) // 128
EPAD = ROWS * 128
WPT = NG // NS
ZCH = 8000


def _sc_body(ei_hbm, attr_hbm, batch_hbm, out_hbm,
             batch_v, src_v, dst_v, attr_v, keys_v, vals_v, zbuf_v, zbuf2_v,
             w_sh, sem_stage, sem_zero, sem_scat, sem_wb):
    c = lax.axis_index("c")
    s = lax.axis_index("s")
    wid = c * NS + s
    ebase = wid * EPT

    cp_b = pltpu.async_copy(batch_hbm, batch_v, sem_stage)
    cp_s = pltpu.async_copy(ei_hbm.at[pl.ds(ebase, EPT)],
                            src_v.at[pl.ds(0, EPT)], sem_stage)
    cp_d = pltpu.async_copy(ei_hbm.at[pl.ds(E + ebase, EPT)],
                            dst_v.at[pl.ds(0, EPT)], sem_stage)
    cp_a = pltpu.async_copy(attr_hbm.at[pl.ds(ebase, EPT)],
                            attr_v.at[pl.ds(0, EPT)], sem_stage)

    def zero_body(i, carry):
        zbuf_v[pl.ds(i * 16, 16)] = jnp.zeros((16,), jnp.float32)
        return carry
    lax.fori_loop(0, ZCH // 16, zero_body, 0)
    zcps = [pltpu.async_copy(zbuf_v, w_sh.at[pl.ds(s * WPT + r * ZCH, ZCH)],
                             sem_zero)
            for r in range(WPT // ZCH)]

    cp_b.wait()
    cp_s.wait()
    cp_d.wait()
    cp_a.wait()

    lanes = lax.iota(jnp.int32, 16)

    def key_body(j, carry):
        for k in range(8):
            off = j * 128 + k * 16
            valid = (off + lanes) < EPT
            d = jnp.where(valid, dst_v[pl.ds(off, 16)], 0)
            sg = plsc.load_gather(batch_v, [d])
            sv = src_v[pl.ds(off, 16)]
            key = jnp.where(valid, sv * G + sg, 0)
            a = jnp.where(valid, attr_v[pl.ds(off, 16)], 0.0)
            keys_v[j, pl.ds(k * 16, 16)] = key
            vals_v[j, pl.ds(k * 16, 16)] = a
        return carry
    lax.fori_loop(0, ROWS, key_body, 0)

    for cp in zcps:
        cp.wait()
    plsc.subcore_barrier()

    def scat_fire(j, carry):
        pltpu.async_copy(vals_v.at[j], w_sh.at[keys_v.at[j]], sem_scat,
                         add=True)
        return carry
    lax.fori_loop(0, ROWS, scat_fire, 0)

    def scat_drain(j, carry):
        pltpu.make_async_copy(vals_v.at[j], w_sh.at[keys_v.at[j]],
                              sem_scat).wait()
        return carry
    lax.fori_loop(0, ROWS, scat_drain, 0)

    plsc.subcore_barrier()

    bufs = (zbuf_v, zbuf2_v)
    outs = []
    for r in range(WPT // ZCH):
        buf = bufs[r % 2]
        if r >= 2:
            outs[r - 2].wait()
        pltpu.sync_copy(w_sh.at[pl.ds(s * WPT + r * ZCH, ZCH)], buf)
        outs.append(pltpu.async_copy(
            buf, out_hbm.at[pl.ds(c * NG + s * WPT + r * ZCH, ZCH)], sem_wb))
    outs[-2].wait()
    outs[-1].wait()


def _build_w(edge_index, attr, batch):
    mesh = plsc.VectorSubcoreMesh(core_axis_name="c", subcore_axis_name="s")
    f = pl.kernel(
        _sc_body,
        out_type=jax.ShapeDtypeStruct((NC * NG,), jnp.float32),
        mesh=mesh,
        compiler_params=pltpu.CompilerParams(needs_layout_passes=False),
        scratch_types=[
            pltpu.VMEM((N,), jnp.int32),
            pltpu.VMEM((EPAD,), jnp.int32),
            pltpu.VMEM((EPAD,), jnp.int32),
            pltpu.VMEM((EPAD,), jnp.float32),
            pltpu.VMEM((ROWS, 128), jnp.int32),
            pltpu.VMEM((ROWS, 128), jnp.float32),
            pltpu.VMEM((ZCH,), jnp.float32),
            pltpu.VMEM((ZCH,), jnp.float32),
            pltpu.VMEM_SHARED((NG,), jnp.float32),
            pltpu.SemaphoreType.DMA,
            pltpu.SemaphoreType.DMA,
            pltpu.SemaphoreType.DMA,
            pltpu.SemaphoreType.DMA,
        ],
    )
    return f(edge_index, attr, batch)


def _bn(z, g, b):
    mu = jnp.mean(z, axis=0, keepdims=True)
    var = jnp.mean((z - mu) * (z - mu), axis=0, keepdims=True)
    return (z - mu) * jax.lax.rsqrt(var + 1e-5) * g + b


def _tc_body(w2_ref, x_ref, batchT_ref,
             fc1_W_ref, fc1_b_ref, bn1_g_ref, bn1_b_ref,
             fc2_W_ref, fc2_b_ref, bn2_g_ref, bn2_b_ref,
             r1_W_ref, r1_b_ref, rbn1_g_ref, rbn1_b_ref,
             r2_W_ref, r2_b_ref, rbn2_g_ref, rbn2_b_ref,
             out_WT_ref, out_b_ref, o_ref):
    hi = jax.lax.Precision.HIGHEST
    w = w2_ref[0] + w2_ref[1]
    sums = jax.lax.dot_general(w, x_ref[...],
                               (((0,), (0,)), ((), ())),
                               precision=hi)
    gids = jax.lax.broadcasted_iota(jnp.int32, (G, N), 0)
    onehot = (gids == batchT_ref[...]).astype(jnp.float32)
    cnt = jnp.sum(onehot, axis=1, keepdims=True)
    p = sums / jnp.maximum(cnt, 1.0)

    h = jnp.maximum(_bn(jnp.dot(p, fc1_W_ref[...], precision=hi)
                        + fc1_b_ref[...], bn1_g_ref[...], bn1_b_ref[...]), 0.0)
    h = jnp.maximum(_bn(jnp.dot(h, fc2_W_ref[...], precision=hi)
                        + fc2_b_ref[...], bn2_g_ref[...], bn2_b_ref[...]), 0.0)
    res = h
    h = _bn(jnp.dot(h, r1_W_ref[...], precision=hi)
            + r1_b_ref[...], rbn1_g_ref[...], rbn1_b_ref[...])
    h = jnp.maximum(h, 0.0)
    h = _bn(jnp.dot(h, r2_W_ref[...], precision=hi)
            + r2_b_ref[...], rbn2_g_ref[...], rbn2_b_ref[...])
    h = jnp.maximum(h + res, 0.0)
    o_ref[...] = (jnp.sum(h * out_WT_ref[...], axis=1, keepdims=True)
                  + out_b_ref[...])


def kernel(x, edge_index, edge_attr, batch,
           fc1_W, fc1_b, bn1_g, bn1_b,
           fc2_W, fc2_b, bn2_g, bn2_b,
           r1_W, r1_b, rbn1_g, rbn1_b,
           r2_W, r2_b, rbn2_g, rbn2_b,
           out_W, out_b):
    attr = edge_attr.reshape(E)
    w_flat = _build_w(edge_index.reshape(2 * E), attr, batch)
    w2 = w_flat.reshape(NC, N, G)

    row = lambda v: v.reshape(1, -1)
    tc = pl.pallas_call(
        _tc_body,
        out_shape=jax.ShapeDtypeStruct((G, 1), jnp.float32),
    )
    return tc(w2, x, batch.reshape(1, N),
              fc1_W, row(fc1_b), row(bn1_g), row(bn1_b),
              fc2_W, row(fc2_b), row(bn2_g), row(bn2_b),
              r1_W, row(r1_b), row(rbn1_g), row(rbn1_b),
              r2_W, row(r2_b), row(rbn2_g), row(rbn2_b),
              out_W.reshape(1, RD), out_b.reshape(1, 1))


# speedup vs baseline: 28.4581x; 1.0064x over previous
import jax
import jax.numpy as jnp
from jax import lax
from jax.experimental import pallas as pl
from jax.experimental.pallas import tpu as pltpu
from jax.experimental.pallas import tpu_sc as plsc

N = 10000
E = 320000
D = 128
G = 64
RD = 256
NG = N * G
NC = 2
NS = 16
NW = NC * NS
EPT = E // NW
ROWS = (EPT + 127) // 128
EPAD = ROWS * 128
WPT = NG // NS
ZCH = 8000


def _sc_body(ei_hbm, attr_hbm, batch_hbm, out_hbm,
             batch_v, src_v, dst_v, attr_v, keys_v, vals_v, zbuf_v, zbuf2_v,
             w_sh, sem_stage, sem_zero, sem_scat, sem_wb):
    c = lax.axis_index("c")
    s = lax.axis_index("s")
    wid = c * NS + s
    ebase = wid * EPT

    cp_b = pltpu.async_copy(batch_hbm, batch_v, sem_stage)
    cp_s = pltpu.async_copy(ei_hbm.at[pl.ds(ebase, EPT)],
                            src_v.at[pl.ds(0, EPT)], sem_stage)
    cp_d = pltpu.async_copy(ei_hbm.at[pl.ds(E + ebase, EPT)],
                            dst_v.at[pl.ds(0, EPT)], sem_stage)
    cp_a = pltpu.async_copy(attr_hbm.at[pl.ds(ebase, EPT)],
                            attr_v.at[pl.ds(0, EPT)], sem_stage)

    def zero_body(i, carry):
        zbuf_v[pl.ds(i * 16, 16)] = jnp.zeros((16,), jnp.float32)
        return carry
    lax.fori_loop(0, ZCH // 16, zero_body, 0)
    zcps = [pltpu.async_copy(zbuf_v, w_sh.at[pl.ds(s * WPT + r * ZCH, ZCH)],
                             sem_zero)
            for r in range(WPT // ZCH)]

    cp_b.wait()
    cp_s.wait()
    cp_d.wait()
    cp_a.wait()

    lanes = lax.iota(jnp.int32, 16)

    def key_body(j, carry):
        for k in range(8):
            off = j * 128 + k * 16
            valid = (off + lanes) < EPT
            d = jnp.where(valid, dst_v[pl.ds(off, 16)], 0)
            sg = plsc.load_gather(batch_v, [d])
            sv = src_v[pl.ds(off, 16)]
            key = jnp.where(valid, sv * G + sg, 0)
            a = jnp.where(valid, attr_v[pl.ds(off, 16)], 0.0)
            keys_v[j, pl.ds(k * 16, 16)] = key
            vals_v[j, pl.ds(k * 16, 16)] = a
        return carry
    lax.fori_loop(0, ROWS, key_body, 0)

    for cp in zcps:
        cp.wait()
    plsc.subcore_barrier()

    def scat_fire(j, carry):
        pltpu.async_copy(vals_v.at[j], w_sh.at[keys_v.at[j]], sem_scat,
                         add=True)
        return carry
    lax.fori_loop(0, ROWS, scat_fire, 0)

    def scat_drain(j, carry):
        pltpu.make_async_copy(vals_v.at[j], w_sh.at[keys_v.at[j]],
                              sem_scat).wait()
        return carry
    lax.fori_loop(0, ROWS, scat_drain, 0)

    plsc.subcore_barrier()

    bufs = (zbuf_v, zbuf2_v)
    outs = []
    for r in range(WPT // ZCH):
        buf = bufs[r % 2]
        if r >= 2:
            outs[r - 2].wait()
        pltpu.sync_copy(w_sh.at[pl.ds(s * WPT + r * ZCH, ZCH)], buf)
        outs.append(pltpu.async_copy(
            buf, out_hbm.at[pl.ds(c * NG + s * WPT + r * ZCH, ZCH)], sem_wb))
    outs[-2].wait()
    outs[-1].wait()


def _build_w(edge_index, attr, batch):
    mesh = plsc.VectorSubcoreMesh(core_axis_name="c", subcore_axis_name="s")
    f = pl.kernel(
        _sc_body,
        out_type=jax.ShapeDtypeStruct((NC * NG,), jnp.float32),
        mesh=mesh,
        compiler_params=pltpu.CompilerParams(needs_layout_passes=False),
        scratch_types=[
            pltpu.VMEM((N,), jnp.int32),
            pltpu.VMEM((EPAD,), jnp.int32),
            pltpu.VMEM((EPAD,), jnp.int32),
            pltpu.VMEM((EPAD,), jnp.float32),
            pltpu.VMEM((ROWS, 128), jnp.int32),
            pltpu.VMEM((ROWS, 128), jnp.float32),
            pltpu.VMEM((ZCH,), jnp.float32),
            pltpu.VMEM((ZCH,), jnp.float32),
            pltpu.VMEM_SHARED((NG,), jnp.float32),
            pltpu.SemaphoreType.DMA,
            pltpu.SemaphoreType.DMA,
            pltpu.SemaphoreType.DMA,
            pltpu.SemaphoreType.DMA,
        ],
    )
    return f(edge_index, attr, batch)


def _bn(z, g, b):
    mu = jnp.mean(z, axis=0, keepdims=True)
    var = jnp.mean((z - mu) * (z - mu), axis=0, keepdims=True)
    return (z - mu) * jax.lax.rsqrt(var + 1e-5) * g + b


def _tc_body(w2_ref, x_ref, batchT_ref,
             fc1_W_ref, fc1_b_ref, bn1_g_ref, bn1_b_ref,
             fc2_W_ref, fc2_b_ref, bn2_g_ref, bn2_b_ref,
             r1_W_ref, r1_b_ref, rbn1_g_ref, rbn1_b_ref,
             r2_W_ref, r2_b_ref, rbn2_g_ref, rbn2_b_ref,
             out_WT_ref, out_b_ref, o_ref):
    hi = jax.lax.Precision.HIGHEST
    w = w2_ref[0] + w2_ref[1]
    sums = jax.lax.dot_general(w, x_ref[...],
                               (((0,), (0,)), ((), ())),
                               precision=hi)
    gids = jax.lax.broadcasted_iota(jnp.int32, (G, N), 0)
    onehot = (gids == batchT_ref[...]).astype(jnp.float32)
    cnt = jnp.sum(onehot, axis=1, keepdims=True)
    p = sums / jnp.maximum(cnt, 1.0)

    h = jnp.maximum(_bn(jnp.dot(p, fc1_W_ref[...])
                        + fc1_b_ref[...], bn1_g_ref[...], bn1_b_ref[...]), 0.0)
    h = jnp.maximum(_bn(jnp.dot(h, fc2_W_ref[...])
                        + fc2_b_ref[...], bn2_g_ref[...], bn2_b_ref[...]), 0.0)
    res = h
    h = _bn(jnp.dot(h, r1_W_ref[...])
            + r1_b_ref[...], rbn1_g_ref[...], rbn1_b_ref[...])
    h = jnp.maximum(h, 0.0)
    h = _bn(jnp.dot(h, r2_W_ref[...])
            + r2_b_ref[...], rbn2_g_ref[...], rbn2_b_ref[...])
    h = jnp.maximum(h + res, 0.0)
    o_ref[...] = (jnp.sum(h * out_WT_ref[...], axis=1, keepdims=True)
                  + out_b_ref[...])


def kernel(x, edge_index, edge_attr, batch,
           fc1_W, fc1_b, bn1_g, bn1_b,
           fc2_W, fc2_b, bn2_g, bn2_b,
           r1_W, r1_b, rbn1_g, rbn1_b,
           r2_W, r2_b, rbn2_g, rbn2_b,
           out_W, out_b):
    attr = edge_attr.reshape(E)
    w_flat = _build_w(edge_index.reshape(2 * E), attr, batch)
    w2 = w_flat.reshape(NC, N, G)

    row = lambda v: v.reshape(1, -1)
    tc = pl.pallas_call(
        _tc_body,
        out_shape=jax.ShapeDtypeStruct((G, 1), jnp.float32),
    )
    return tc(w2, x, batch.reshape(1, N),
              fc1_W, row(fc1_b), row(bn1_g), row(bn1_b),
              fc2_W, row(fc2_b), row(bn2_g), row(bn2_b),
              r1_W, row(r1_b), row(rbn1_g), row(rbn1_b),
              r2_W, row(r2_b), row(rbn2_g), row(rbn2_b),
              out_W.reshape(1, RD), out_b.reshape(1, 1))


# speedup vs baseline: 35.0626x; 1.2321x over previous
import jax
import jax.numpy as jnp
from jax import lax
from jax.experimental import pallas as pl
from jax.experimental.pallas import tpu as pltpu
from jax.experimental.pallas import tpu_sc as plsc

N = 10000
E = 320000
D = 128
G = 64
H1 = 256
RD = 256
NG = N * G
NC = 2
NS = 16
NW = NC * NS
EPT = E // NW
ROWS = (EPT + 127) // 128
EPAD = ROWS * 128
WPT = NG // NS
ZCH = 8000


def _sc_body(ei_hbm, attr_hbm, batch_hbm, out_hbm,
             batch_v, src_v, dst_v, attr_v, keys_v, vals_v, zbuf_v, zbuf2_v,
             w_sh, sem_stage, sem_zero, sem_scat, sem_wb):
    c = lax.axis_index("c")
    s = lax.axis_index("s")
    wid = c * NS + s
    ebase = wid * EPT

    cp_b = pltpu.async_copy(batch_hbm, batch_v, sem_stage)
    cp_s = pltpu.async_copy(ei_hbm.at[pl.ds(ebase, EPT)],
                            src_v.at[pl.ds(0, EPT)], sem_stage)
    cp_d = pltpu.async_copy(ei_hbm.at[pl.ds(E + ebase, EPT)],
                            dst_v.at[pl.ds(0, EPT)], sem_stage)
    cp_a = pltpu.async_copy(attr_hbm.at[pl.ds(ebase, EPT)],
                            attr_v.at[pl.ds(0, EPT)], sem_stage)

    def zero_body(i, carry):
        zbuf_v[pl.ds(i * 16, 16)] = jnp.zeros((16,), jnp.float32)
        return carry
    lax.fori_loop(0, ZCH // 16, zero_body, 0)
    zcps = [pltpu.async_copy(zbuf_v, w_sh.at[pl.ds(s * WPT + r * ZCH, ZCH)],
                             sem_zero)
            for r in range(WPT // ZCH)]

    for cp in zcps:
        cp.wait()
    plsc.subcore_barrier()

    cp_b.wait()
    cp_s.wait()
    cp_d.wait()
    cp_a.wait()

    lanes = lax.iota(jnp.int32, 16)

    def key_body(j, carry):
        for k in range(8):
            off = j * 128 + k * 16
            valid = (off + lanes) < EPT
            d = jnp.where(valid, dst_v[pl.ds(off, 16)], 0)
            sg = plsc.load_gather(batch_v, [d])
            sv = src_v[pl.ds(off, 16)]
            key = jnp.where(valid, sv * G + sg, 0)
            a = jnp.where(valid, attr_v[pl.ds(off, 16)], 0.0)
            keys_v[j, pl.ds(k * 16, 16)] = key
            vals_v[j, pl.ds(k * 16, 16)] = a
        return carry
    lax.fori_loop(0, ROWS, key_body, 0)

    def scat_fire(j, carry):
        pltpu.async_copy(vals_v.at[j], w_sh.at[keys_v.at[j]], sem_scat,
                         add=True)
        return carry
    lax.fori_loop(0, ROWS, scat_fire, 0)

    def scat_drain(j, carry):
        pltpu.make_async_copy(vals_v.at[j], w_sh.at[keys_v.at[j]],
                              sem_scat).wait()
        return carry
    lax.fori_loop(0, ROWS, scat_drain, 0)

    plsc.subcore_barrier()

    bufs = (zbuf_v, zbuf2_v)
    outs = []
    for r in range(WPT // ZCH):
        buf = bufs[r % 2]
        if r >= 2:
            outs[r - 2].wait()
        pltpu.sync_copy(w_sh.at[pl.ds(s * WPT + r * ZCH, ZCH)], buf)
        outs.append(pltpu.async_copy(
            buf, out_hbm.at[pl.ds(c * NG + s * WPT + r * ZCH, ZCH)], sem_wb))
    outs[-2].wait()
    outs[-1].wait()


def _build_w(edge_index, attr, batch):
    mesh = plsc.VectorSubcoreMesh(core_axis_name="c", subcore_axis_name="s")
    f = pl.kernel(
        _sc_body,
        out_type=jax.ShapeDtypeStruct((NC * NG,), jnp.float32),
        mesh=mesh,
        compiler_params=pltpu.CompilerParams(needs_layout_passes=False),
        scratch_types=[
            pltpu.VMEM((N,), jnp.int32),
            pltpu.VMEM((EPAD,), jnp.int32),
            pltpu.VMEM((EPAD,), jnp.int32),
            pltpu.VMEM((EPAD,), jnp.float32),
            pltpu.VMEM((ROWS, 128), jnp.int32),
            pltpu.VMEM((ROWS, 128), jnp.float32),
            pltpu.VMEM((ZCH,), jnp.float32),
            pltpu.VMEM((ZCH,), jnp.float32),
            pltpu.VMEM_SHARED((NG,), jnp.float32),
            pltpu.SemaphoreType.DMA,
            pltpu.SemaphoreType.DMA,
            pltpu.SemaphoreType.DMA,
            pltpu.SemaphoreType.DMA,
        ],
    )
    return f(edge_index, attr, batch)


def _bn(z, g, b):
    mu = jnp.mean(z, axis=0, keepdims=True)
    var = jnp.mean((z - mu) * (z - mu), axis=0, keepdims=True)
    return (z - mu) * jax.lax.rsqrt(var + 1e-5) * g + b


BN = 2000
NSTEPS = N // BN


def _tc_body(w2_ref, x_ref, batchT_ref,
             fc1_W_ref, fc1_b_ref, bn1_g_ref, bn1_b_ref,
             fc2_W_ref, fc2_b_ref, bn2_g_ref, bn2_b_ref,
             r1_W_ref, r1_b_ref, rbn1_g_ref, rbn1_b_ref,
             r2_W_ref, r2_b_ref, rbn2_g_ref, rbn2_b_ref,
             out_WT_ref, out_b_ref, o_ref, sums_acc, cnt_acc):
    hi = jax.lax.Precision.HIGHEST
    i = pl.program_id(0)

    w = w2_ref[0] + w2_ref[1]
    psum = jax.lax.dot_general(w, x_ref[...],
                               (((0,), (0,)), ((), ())),
                               precision=hi)
    gids = jax.lax.broadcasted_iota(jnp.int32, (G, BN), 0)
    onehot = (gids == batchT_ref[0]).astype(jnp.float32)
    pcnt = jnp.sum(onehot, axis=1, keepdims=True)

    @pl.when(i == 0)
    def _():
        sums_acc[...] = psum
        cnt_acc[...] = pcnt

    @pl.when(i > 0)
    def _():
        sums_acc[...] += psum
        cnt_acc[...] += pcnt

    @pl.when(i == NSTEPS - 1)
    def _():
        p = sums_acc[...] / jnp.maximum(cnt_acc[...], 1.0)
        h = jnp.maximum(_bn(jnp.dot(p, fc1_W_ref[...]) + fc1_b_ref[...],
                            bn1_g_ref[...], bn1_b_ref[...]), 0.0)
        h = jnp.maximum(_bn(jnp.dot(h, fc2_W_ref[...]) + fc2_b_ref[...],
                            bn2_g_ref[...], bn2_b_ref[...]), 0.0)
        res = h
        h = _bn(jnp.dot(h, r1_W_ref[...]) + r1_b_ref[...],
                rbn1_g_ref[...], rbn1_b_ref[...])
        h = jnp.maximum(h, 0.0)
        h = _bn(jnp.dot(h, r2_W_ref[...]) + r2_b_ref[...],
                rbn2_g_ref[...], rbn2_b_ref[...])
        h = jnp.maximum(h + res, 0.0)
        o_ref[...] = (jnp.sum(h * out_WT_ref[...], axis=1, keepdims=True)
                      + out_b_ref[...])


def kernel(x, edge_index, edge_attr, batch,
           fc1_W, fc1_b, bn1_g, bn1_b,
           fc2_W, fc2_b, bn2_g, bn2_b,
           r1_W, r1_b, rbn1_g, rbn1_b,
           r2_W, r2_b, rbn2_g, rbn2_b,
           out_W, out_b):
    attr = edge_attr.reshape(E)
    w_flat = _build_w(edge_index.reshape(2 * E), attr, batch)
    w2 = w_flat.reshape(NC, N, G)

    row = lambda v: v.reshape(1, -1)
    full = lambda *shape: pl.BlockSpec(shape, lambda i: (0,) * len(shape))
    tc = pl.pallas_call(
        _tc_body,
        grid=(NSTEPS,),
        in_specs=[
            pl.BlockSpec((NC, BN, G), lambda i: (0, i, 0)),
            pl.BlockSpec((BN, D), lambda i: (i, 0)),
            pl.BlockSpec((1, 1, BN), lambda i: (i, 0, 0)),
            full(D, H1), full(1, H1), full(1, H1), full(1, H1),
            full(H1, RD), full(1, RD), full(1, RD), full(1, RD),
            full(RD, RD), full(1, RD), full(1, RD), full(1, RD),
            full(RD, RD), full(1, RD), full(1, RD), full(1, RD),
            full(1, RD), full(1, 1),
        ],
        out_specs=pl.BlockSpec((G, 1), lambda i: (0, 0)),
        scratch_shapes=[
            pltpu.VMEM((G, D), jnp.float32),
            pltpu.VMEM((G, 1), jnp.float32),
        ],
        out_shape=jax.ShapeDtypeStruct((G, 1), jnp.float32),
    )
    return tc(w2, x, batch.reshape(NSTEPS, 1, BN),
              fc1_W, row(fc1_b), row(bn1_g), row(bn1_b),
              fc2_W, row(fc2_b), row(bn2_g), row(bn2_b),
              r1_W, row(r1_b), row(rbn1_g), row(rbn1_b),
              r2_W, row(r2_b), row(rbn2_g), row(rbn2_b),
              out_W.reshape(1, RD), out_b.reshape(1, 1))


# speedup vs baseline: 42.0591x; 1.1995x over previous
import jax
import jax.numpy as jnp
from jax import lax
from jax.experimental import pallas as pl
from jax.experimental.pallas import tpu as pltpu
from jax.experimental.pallas import tpu_sc as plsc

N = 10000
E = 320000
D = 128
G = 64
H1 = 256
RD = 256
NG = N * G
NC = 2
NS = 16
NW = NC * NS
TROW = E // 128
ROWS = TROW // NW + 1
EPAD = ROWS * 128
WPT = NG // NS
ZCH = 8000


def _sc_body(ei_hbm, attr_hbm, batch_hbm, out_hbm,
             batch_v, ei_v, attr_v, keys_v, vals_v, zbuf_v, zbuf2_v,
             w_sh, sem_stage, sem_zero, sem_scat, sem_wb):
    c = lax.axis_index("c")
    s = lax.axis_index("s")
    wid = c * NS + s
    r0 = (TROW * wid) // NW
    nr = (TROW * (wid + 1)) // NW - r0

    cp_b = pltpu.async_copy(batch_hbm, batch_v, sem_stage)
    cp_e = pltpu.async_copy(ei_hbm.at[:, pl.ds(r0 * 128, EPAD)],
                            ei_v, sem_stage)
    cp_a = pltpu.async_copy(attr_hbm.at[pl.ds(r0, ROWS)], attr_v, sem_stage)

    def zero_body(i, carry):
        zbuf_v[pl.ds(i * 16, 16)] = jnp.zeros((16,), jnp.float32)
        return carry
    lax.fori_loop(0, ZCH // 16, zero_body, 0)
    zcps = [pltpu.async_copy(zbuf_v, w_sh.at[pl.ds(s * WPT + r * ZCH, ZCH)],
                             sem_zero)
            for r in range(WPT // ZCH)]

    for cp in zcps:
        cp.wait()
    plsc.subcore_barrier()

    cp_b.wait()
    cp_e.wait()
    cp_a.wait()

    def key_body(j, carry):
        row_valid = j < nr
        for k in range(8):
            off = j * 128 + k * 16
            d = jnp.where(row_valid, ei_v[1, pl.ds(off, 16)], 0)
            sg = plsc.load_gather(batch_v, [d])
            sv = ei_v[0, pl.ds(off, 16)]
            hi_half = (sv >= N // 2).astype(jnp.int32)
            key = (sv - (N // 2) * hi_half) * (2 * G) + G * hi_half + sg
            keys_v[j, pl.ds(k * 16, 16)] = jnp.where(row_valid, key, 0)
            vals_v[j, pl.ds(k * 16, 16)] = jnp.where(
                row_valid, attr_v[j, 0, pl.ds(k * 16, 16)], 0.0)
        return carry
    lax.fori_loop(0, ROWS, key_body, 0)

    def scat_fire(j, carry):
        pltpu.async_copy(vals_v.at[j], w_sh.at[keys_v.at[j]], sem_scat,
                         add=True)
        return carry
    lax.fori_loop(0, ROWS, scat_fire, 0)

    def scat_drain(j, carry):
        pltpu.make_async_copy(vals_v.at[j], w_sh.at[keys_v.at[j]],
                              sem_scat).wait()
        return carry
    lax.fori_loop(0, ROWS, scat_drain, 0)

    plsc.subcore_barrier()

    bufs = (zbuf_v, zbuf2_v)
    outs = []
    for r in range(WPT // ZCH):
        buf = bufs[r % 2]
        if r >= 2:
            outs[r - 2].wait()
        pltpu.sync_copy(w_sh.at[pl.ds(s * WPT + r * ZCH, ZCH)], buf)
        outs.append(pltpu.async_copy(
            buf, out_hbm.at[pl.ds(c * NG + s * WPT + r * ZCH, ZCH)], sem_wb))
    outs[-2].wait()
    outs[-1].wait()


def _build_w(edge_index, attr, batch):
    mesh = plsc.VectorSubcoreMesh(core_axis_name="c", subcore_axis_name="s")
    f = pl.kernel(
        _sc_body,
        out_type=jax.ShapeDtypeStruct((NC * NG,), jnp.float32),
        mesh=mesh,
        compiler_params=pltpu.CompilerParams(needs_layout_passes=False),
        scratch_types=[
            pltpu.VMEM((N,), jnp.int32),
            pltpu.VMEM((2, EPAD), jnp.int32),
            pltpu.VMEM((ROWS, 1, 128), jnp.float32),
            pltpu.VMEM((ROWS, 128), jnp.int32),
            pltpu.VMEM((ROWS, 128), jnp.float32),
            pltpu.VMEM((ZCH,), jnp.float32),
            pltpu.VMEM((ZCH,), jnp.float32),
            pltpu.VMEM_SHARED((NG,), jnp.float32),
            pltpu.SemaphoreType.DMA,
            pltpu.SemaphoreType.DMA,
            pltpu.SemaphoreType.DMA,
            pltpu.SemaphoreType.DMA,
        ],
    )
    return f(edge_index, attr, batch)


def _bn(z, g, b):
    mu = jnp.mean(z, axis=0, keepdims=True)
    var = jnp.mean((z - mu) * (z - mu), axis=0, keepdims=True)
    return (z - mu) * jax.lax.rsqrt(var + 1e-5) * g + b


BN = 2000
NSTEPS = N // BN


def _tc_body(w2_ref, xlo_ref, xhi_ref, batchT_ref,
             fc1_W_ref, fc1_b_ref, bn1_g_ref, bn1_b_ref,
             fc2_W_ref, fc2_b_ref, bn2_g_ref, bn2_b_ref,
             r1_W_ref, r1_b_ref, rbn1_g_ref, rbn1_b_ref,
             r2_W_ref, r2_b_ref, rbn2_g_ref, rbn2_b_ref,
             out_WT_ref, out_b_ref, o_ref, sums_acc, cnt_acc):
    hi = jax.lax.Precision.HIGHEST
    i = pl.program_id(0)

    w = w2_ref[0] + w2_ref[1]
    s1 = jax.lax.dot_general(w, xlo_ref[...],
                             (((0,), (0,)), ((), ())), precision=hi)
    s2 = jax.lax.dot_general(w, xhi_ref[...],
                             (((0,), (0,)), ((), ())), precision=hi)
    psum = s1[:G] + s2[G:]
    gids = jax.lax.broadcasted_iota(jnp.int32, (G, BN), 0)
    onehot = (gids == batchT_ref[0]).astype(jnp.float32)
    pcnt = jnp.sum(onehot, axis=1, keepdims=True)

    @pl.when(i == 0)
    def _():
        sums_acc[...] = psum
        cnt_acc[...] = pcnt

    @pl.when(i > 0)
    def _():
        sums_acc[...] += psum
        cnt_acc[...] += pcnt

    @pl.when(i == NSTEPS - 1)
    def _():
        p = sums_acc[...] / jnp.maximum(cnt_acc[...], 1.0)
        h = jnp.maximum(_bn(jnp.dot(p, fc1_W_ref[...]) + fc1_b_ref[...],
                            bn1_g_ref[...], bn1_b_ref[...]), 0.0)
        h = jnp.maximum(_bn(jnp.dot(h, fc2_W_ref[...]) + fc2_b_ref[...],
                            bn2_g_ref[...], bn2_b_ref[...]), 0.0)
        res = h
        h = _bn(jnp.dot(h, r1_W_ref[...]) + r1_b_ref[...],
                rbn1_g_ref[...], rbn1_b_ref[...])
        h = jnp.maximum(h, 0.0)
        h = _bn(jnp.dot(h, r2_W_ref[...]) + r2_b_ref[...],
                rbn2_g_ref[...], rbn2_b_ref[...])
        h = jnp.maximum(h + res, 0.0)
        o_ref[...] = (jnp.sum(h * out_WT_ref[...], axis=1, keepdims=True)
                      + out_b_ref[...]).reshape(1, G)


def kernel(x, edge_index, edge_attr, batch,
           fc1_W, fc1_b, bn1_g, bn1_b,
           fc2_W, fc2_b, bn2_g, bn2_b,
           r1_W, r1_b, rbn1_g, rbn1_b,
           r2_W, r2_b, rbn2_g, rbn2_b,
           out_W, out_b):
    w_flat = _build_w(edge_index, edge_attr.reshape(TROW, 1, 128), batch)
    w2 = w_flat.reshape(NC, N // 2, 2 * G)

    row = lambda v: v.reshape(1, -1)
    full = lambda *shape: pl.BlockSpec(shape, lambda i: (0,) * len(shape))
    tc = pl.pallas_call(
        _tc_body,
        grid=(NSTEPS,),
        in_specs=[
            pl.BlockSpec((NC, BN // 2, 2 * G), lambda i: (0, i, 0)),
            pl.BlockSpec((BN // 2, D), lambda i: (i, 0)),
            pl.BlockSpec((BN // 2, D), lambda i: (i + NSTEPS, 0)),
            pl.BlockSpec((1, 1, BN), lambda i: (i, 0, 0)),
            full(D, H1), full(1, H1), full(1, H1), full(1, H1),
            full(H1, RD), full(1, RD), full(1, RD), full(1, RD),
            full(RD, RD), full(1, RD), full(1, RD), full(1, RD),
            full(RD, RD), full(1, RD), full(1, RD), full(1, RD),
            full(1, RD), full(1, 1),
        ],
        out_specs=pl.BlockSpec((1, G), lambda i: (0, 0)),
        scratch_shapes=[
            pltpu.VMEM((G, D), jnp.float32),
            pltpu.VMEM((G, 1), jnp.float32),
        ],
        out_shape=jax.ShapeDtypeStruct((1, G), jnp.float32),
    )
    out = tc(w2, x, x, batch.reshape(NSTEPS, 1, BN),
              fc1_W, row(fc1_b), row(bn1_g), row(bn1_b),
              fc2_W, row(fc2_b), row(bn2_g), row(bn2_b),
              r1_W, row(r1_b), row(rbn1_g), row(rbn1_b),
              r2_W, row(r2_b), row(rbn2_g), row(rbn2_b),
              out_W.reshape(1, RD), out_b.reshape(1, 1))
    return out.reshape(G, 1)
